# Initial kernel scaffold; baseline (speedup 1.0000x reference)
import jax
import jax.numpy as jnp
from jax.experimental import pallas as pl


def kernel(p_scores, indices, relation_mask, entities):
    raise NotImplementedError("write your pallas kernel here")


# speedup vs baseline: 148.9774x; 148.9774x over previous
import functools

import jax
import jax.numpy as jnp
from jax import lax
from jax.experimental import pallas as pl
from jax.experimental.pallas import tpu as pltpu
from jax.experimental.pallas import tpu_sc as plsc

_NC = 2
_NS = 16
_L = 16

_ROWS = 80
_LANES = 128


def _build_sc_call(num_entities, num_relations, num_edges):
    nw = _NC * _NS
    chunk = num_edges // nw
    groups = chunk // _L

    mesh = plsc.VectorSubcoreMesh(core_axis_name="c", subcore_axis_name="s")

    @functools.partial(
        pl.kernel,
        out_type=jax.ShapeDtypeStruct((_NC, _ROWS, _LANES), jnp.float32),
        mesh=mesh,
        compiler_params=pltpu.CompilerParams(needs_layout_passes=False),
        scratch_types=[
            pltpu.VMEM((chunk,), jnp.int32),
            pltpu.VMEM((chunk,), jnp.int32),
            pltpu.VMEM((chunk,), jnp.int32),
            pltpu.VMEM((num_entities,), jnp.float32),
            pltpu.VMEM((num_relations,), jnp.float32),
            pltpu.VMEM((_ROWS, _LANES), jnp.float32),
            pltpu.VMEM((_ROWS,), jnp.int32),
            pltpu.VMEM_SHARED((_ROWS, _LANES), jnp.float32),
        ],
    )
    def mp_kernel(rows_hbm, cols_hbm, rel_hbm, ent_hbm, ps_hbm, zeros_hbm,
                  rowids_hbm, out_hbm,
                  rows_v, cols_v, rel_v, ent_v, ps_v, acc_v, rowids_v,
                  shared_acc):
        cid = lax.axis_index("c")
        sid = lax.axis_index("s")
        wid = cid * _NS + sid
        base = wid * chunk

        pltpu.sync_copy(rows_hbm.at[pl.ds(base, chunk)], rows_v)
        pltpu.sync_copy(cols_hbm.at[pl.ds(base, chunk)], cols_v)
        pltpu.sync_copy(rel_hbm.at[pl.ds(base, chunk)], rel_v)
        pltpu.sync_copy(ent_hbm, ent_v)
        pltpu.sync_copy(ps_hbm, ps_v)
        pltpu.sync_copy(zeros_hbm, acc_v)
        pltpu.sync_copy(rowids_hbm, rowids_v)

        @pl.when(sid == 0)
        def _():
            pltpu.sync_copy(zeros_hbm, shared_acc)

        def body(i, carry):
            off = i * _L
            r = rows_v[pl.ds(off, _L)]
            c = cols_v[pl.ds(off, _L)]
            m = rel_v[pl.ds(off, _L)]
            p = plsc.load_gather(ps_v, [m])
            e = plsc.load_gather(ent_v, [r])
            j = lax.rem(c, num_entities)
            hi = lax.shift_right_logical(j, 7)
            lo = lax.bitwise_and(j, _LANES - 1)
            plsc.addupdate_scatter(acc_v, [hi, lo], p * e)
            return carry

        lax.fori_loop(0, groups, body, 0)

        plsc.subcore_barrier()
        pltpu.sync_copy(acc_v, shared_acc.at[rowids_v], add=True)
        plsc.subcore_barrier()

        @pl.when(sid == 0)
        def _():
            pltpu.sync_copy(shared_acc, out_hbm.at[cid])

    return mp_kernel


def kernel(p_scores, indices, relation_mask, entities):
    num_entities = entities.shape[0]
    num_relations = p_scores.shape[0]
    num_edges = indices.shape[1]

    rows = indices[0]
    cols = indices[1]
    ent_flat = entities[:, 0]
    zeros2d = jnp.zeros((_ROWS, _LANES), jnp.float32)
    rowids = jnp.arange(_ROWS, dtype=jnp.int32)

    sc_call = _build_sc_call(num_entities, num_relations, num_edges)
    partials = sc_call(rows, cols, relation_mask, ent_flat, p_scores,
                       zeros2d, rowids)
    y = partials.sum(axis=0).reshape(_ROWS * _LANES)[:num_entities]
    return (y, num_entities)
